# baseline (speedup 1.0000x reference)
import jax
import jax.numpy as jnp
from jax import lax
from jax.experimental import pallas as pl
from jax.experimental.pallas import tpu as pltpu
from jax.experimental.pallas import tpu_sc as plsc

N = 10000
F = 128
NR = 16
W = 160
E_TOT = 200000
E_PAD = 204800
NC, NS = 2, 16
NPAD = 10112
PER_TILE = E_PAD // (NC * NS)
CHUNK = 64
NCHUNK = PER_TILE // CHUNK
NCH0 = 140
NCH1 = 60
NGRP = CHUNK // 16
RPT = NPAD // NS
ALPHA = 0.2


def _pre_body(x_ref, aT_ref, a2T_ref, y1_ref, y2e_ref, s1_ref, u3_ref):
    x = x_ref[...]
    aT1 = aT_ref[0:F, :]
    aT2 = aT_ref[F:2 * F, :]
    aT3 = aT_ref[2 * F:2 * F + NR, :]
    a2T = a2T_ref[...]
    y1 = jnp.dot(x, aT1, preferred_element_type=jnp.float32)
    y2 = jnp.dot(x, aT2, preferred_element_type=jnp.float32)
    y1_ref[...] = y1
    y2e_ref[:, 0:F] = y2
    s2 = jnp.dot(y2, a2T, preferred_element_type=jnp.float32)
    y2e_ref[:, F:W] = jnp.concatenate(
        [s2, jnp.ones((N, 1), jnp.float32),
         jnp.zeros((N, W - F - 2), jnp.float32)], axis=1)
    s1_ref[...] = jnp.dot(y1, a2T, preferred_element_type=jnp.float32)
    u3_ref[...] = jnp.dot(aT3, a2T, preferred_element_type=jnp.float32)


def _dense_pre(x, aT, a2T):
    return pl.pallas_call(
        _pre_body,
        out_shape=[
            jax.ShapeDtypeStruct((N, F), jnp.float32),
            jax.ShapeDtypeStruct((N, W), jnp.float32),
            jax.ShapeDtypeStruct((N, 1), jnp.float32),
            jax.ShapeDtypeStruct((NR, 1), jnp.float32),
        ],
    )(x, aT, a2T)


def _sc_body(sd_h, ee_h, s1_h, u3_h, y2e_h, zacc_h,
             p128_h, p32_h,
             u3_v, sd, six, eeb, pay, sv1, evb, acc,
             sdsem, eesem, rsem, g1sem, ssem):
    c = lax.axis_index("c")
    s = lax.axis_index("s")
    w = c * NS + s

    pltpu.sync_copy(zacc_h.at[pl.ds(s * RPT, RPT)], acc.at[pl.ds(s * RPT, RPT)])
    pltpu.sync_copy(u3_h, u3_v)
    plsc.subcore_barrier()

    u3vec = u3_v[...]
    u3b = [jnp.full((16,), u3vec[r], jnp.float32) for r in range(NR)]
    iota16 = lax.iota(jnp.int32, 16)
    cnt = jnp.where(c == 0, NCH0, NCH1)
    cbase = jnp.where(c == 0, s * NCH0, NS * NCH0 + s * NCH1)

    def issue_meta(j, b):
        jc = jnp.minimum(cbase + j, cbase + cnt - 1)
        pltpu.async_copy(sd_h.at[pl.ds(jc, 1)], sd[b], sdsem[b])
        pltpu.async_copy(ee_h.at[pl.ds(jc * CHUNK * NR, CHUNK * NR)],
                         eeb[b], eesem[b])

    def wait_meta(b):
        pltpu.make_async_copy(sd_h.at[pl.ds(0, 1)], sd[b], sdsem[b]).wait()

    def issue_gathers(b):
        pltpu.async_copy(y2e_h.at[sd[b].at[0].at[1]], pay[b], rsem[b])
        pltpu.async_copy(s1_h.at[sd[b].at[0].at[0]], sv1[b], g1sem[b])

    def wait_scatter(b):
        pltpu.make_async_copy(pay[b], acc.at[six[b].at[0]], ssem[b]).wait()

    def compute(k, b):
        e_base = k * CHUNK
        pltpu.make_async_copy(ee_h.at[pl.ds(0, CHUNK * NR)], eeb[b],
                              eesem[b]).wait()
        for g in range(NGRP):
            eidx = (jnp.full((16,), g * 16, jnp.int32) + iota16) * NR
            t = jnp.zeros((16,), jnp.float32)
            for r in range(NR):
                t = t + u3b[r] * plsc.load_gather(eeb[b], [eidx + r])
            evb[b][pl.ds(g * 16, 16)] = t
        pltpu.make_async_copy(s1_h.at[sd[b].at[0].at[0]], sv1[b],
                              g1sem[b]).wait()
        pltpu.make_async_copy(y2e_h.at[sd[b].at[0].at[1]], pay[b],
                              rsem[b]).wait()
        for g in range(NGRP):
            sl = pl.ds(g * 16, 16)
            eidx16 = jnp.full((16,), g * 16, jnp.int32) + iota16
            s2g = plsc.load_gather(pay[b], [eidx16, jnp.full((16,), F, jnp.int32)])
            pre = sv1[b][sl] + s2g + evb[b][sl]
            ev = jnp.exp(-jnp.maximum(pre, ALPHA * pre))
            gidx = jnp.full((16,), e_base + g * 16, jnp.int32) + iota16
            evb[b][sl] = jnp.where(gidx < E_TOT, ev, 0.0)
            six[b][0, sl] = sd[b][0, 0, sl]

        def grp(g2, carry2):
            evg = evb[b][pl.ds(g2 * 16, 16)]
            base = g2 * 16
            for j in range(16):
                e = base + j
                evv = jnp.full((16,), evg[j], jnp.float32)
                for r in range(9):
                    pay[b][e, pl.ds(r * 16, 16)] = \
                        evv * pay[b][e, pl.ds(r * 16, 16)]
                pay[b][e, pl.ds(F + 16, NR)] = evv * eeb[b][pl.ds(e * NR, NR)]
            return carry2

        lax.fori_loop(0, NGRP, grp, 0)

    def issue_scatter(b):
        pltpu.async_copy(pay[b], acc.at[six[b].at[0]], ssem[b], add=True)

    issue_meta(0, 0)
    issue_meta(1, 1)
    wait_meta(0)
    issue_gathers(0)
    wait_meta(1)
    issue_gathers(1)
    compute(cbase + 0, 0)
    issue_meta(2, 0)
    issue_scatter(0)

    def chunk_pair(kk, carry):
        for par in range(2):
            b = (1 + par) % 2
            k = 2 * kk + 1 + par
            compute(cbase + k, b)
            wait_scatter(1 - b)
            wait_meta(1 - b)
            issue_gathers(1 - b)
            issue_meta(k + 2, b)
            issue_scatter(b)
        return carry

    lax.fori_loop(0, (cnt - 2) // 2, chunk_pair, 0)

    b = 1
    compute(cbase + cnt - 1, b)
    wait_scatter(1 - b)
    issue_scatter(b)
    wait_scatter(b)
    wait_meta(0)
    pltpu.make_async_copy(ee_h.at[pl.ds(0, CHUNK * NR)], eeb[0], eesem[0]).wait()

    plsc.subcore_barrier()
    sl = pl.ds(s * RPT, RPT)
    pltpu.sync_copy(acc.at[sl, pl.ds(0, F)], p128_h.at[c].at[sl])
    pltpu.sync_copy(acc.at[sl, pl.ds(F, W - F)], p32_h.at[c].at[sl])


def _sc_edge(sd3, ee1d, s1, u3, y2e, zacc):
    mesh = plsc.VectorSubcoreMesh(core_axis_name="c", subcore_axis_name="s",
                                  num_cores=NC, num_subcores=NS)
    f = pl.kernel(
        _sc_body,
        out_type=[jax.ShapeDtypeStruct((NC, NPAD, F), jnp.float32),
                  jax.ShapeDtypeStruct((NC, NPAD, W - F), jnp.float32)],
        mesh=mesh,
        compiler_params=pltpu.CompilerParams(needs_layout_passes=False,
                                             use_tc_tiling_on_sc=False),
        scratch_types=[
            pltpu.VMEM((NR,), jnp.float32),
            [pltpu.VMEM((1, 2, CHUNK), jnp.int32) for _ in range(2)],
            [pltpu.VMEM((1, CHUNK), jnp.int32) for _ in range(2)],
            [pltpu.VMEM((CHUNK * NR,), jnp.float32) for _ in range(2)],
            [pltpu.VMEM((CHUNK, W), jnp.float32) for _ in range(2)],
            [pltpu.VMEM((CHUNK,), jnp.float32) for _ in range(2)],
            [pltpu.VMEM((CHUNK,), jnp.float32) for _ in range(2)],
            pltpu.VMEM_SHARED((NPAD, W), jnp.float32),
            [pltpu.SemaphoreType.DMA for _ in range(2)],
            [pltpu.SemaphoreType.DMA for _ in range(2)],
            [pltpu.SemaphoreType.DMA for _ in range(2)],
            [pltpu.SemaphoreType.DMA for _ in range(2)],
            [pltpu.SemaphoreType.DMA for _ in range(2)],
        ],
    )
    return f(sd3, ee1d, s1, u3, y2e, zacc)


def _combine_body(p128_ref, p32_ref, y1_ref, aT_ref, out_ref):
    s2sum = p128_ref[0, 0:N, :] + p128_ref[1, 0:N, :]
    q32 = p32_ref[0, 0:N, :] + p32_ref[1, 0:N, :]
    rs = q32[:, 1:2]
    r16 = q32[:, 16:32]
    aT3 = aT_ref[2 * F:2 * F + NR, :]
    ra = jnp.dot(r16, aT3, preferred_element_type=jnp.float32)
    rssafe = jnp.where(rs > 0, rs, 1.0)
    h = y1_ref[...] + (s2sum + ra) / rssafe
    h = jnp.where(rs > 0, h, 0.0)
    out_ref[...] = jnp.where(h > 0, h, jnp.exp(h) - 1.0)


def _combine(p128, p32, y1, aT):
    return pl.pallas_call(
        _combine_body,
        out_shape=jax.ShapeDtypeStruct((N, F), jnp.float32),
    )(p128, p32, y1, aT)


@jax.jit
def kernel(input, edge, edge_embed, edge_list_nhop, edge_embed_nhop, a, a_2):
    x = input
    aT = a.T
    a2T = a_2.T
    src = jnp.concatenate([edge[0], edge_list_nhop[0],
                           jnp.zeros((E_PAD - E_TOT,), jnp.int32)])
    dst = jnp.concatenate([edge[1], edge_list_nhop[1],
                           jnp.zeros((E_PAD - E_TOT,), jnp.int32)])
    ee = jnp.concatenate([edge_embed, edge_embed_nhop,
                          jnp.zeros((E_PAD - E_TOT, NR), jnp.float32)], axis=0)
    sd3 = jnp.stack([src.reshape(-1, CHUNK), dst.reshape(-1, CHUNK)], axis=1)
    ee1d = ee.reshape(E_PAD * NR)

    y1, y2e, s1, u3 = _dense_pre(x, aT, a2T)
    zacc = jnp.zeros((NPAD, W), jnp.float32)
    p128, p32 = _sc_edge(sd3, ee1d, s1.reshape(N), u3.reshape(NR), y2e, zacc)
    return _combine(p128, p32, y1, aT)


# speedup vs baseline: 1.0463x; 1.0463x over previous
import jax
import jax.numpy as jnp
from jax import lax
from jax.experimental import pallas as pl
from jax.experimental.pallas import tpu as pltpu
from jax.experimental.pallas import tpu_sc as plsc

N = 10000
F = 128
NR = 16
W = 160
E_TOT = 200000
E_PAD = 204800
NC, NS = 2, 16
NPAD = 10112
PER_TILE = E_PAD // (NC * NS)
CHUNK = 64
NCHUNK = PER_TILE // CHUNK
NCH0 = 140
NCH1 = 60
NGRP = CHUNK // 16
RPT = NPAD // NS
ALPHA = 0.2


def _pre_body(x_ref, aT_ref, a2T_ref, y1_ref, y2e_ref, s1_ref, u3_ref):
    x = x_ref[...]
    aT1 = aT_ref[0:F, :]
    aT2 = aT_ref[F:2 * F, :]
    aT3 = aT_ref[2 * F:2 * F + NR, :]
    a2T = a2T_ref[...]
    y1 = jnp.dot(x, aT1, preferred_element_type=jnp.float32)
    y2 = jnp.dot(x, aT2, preferred_element_type=jnp.float32)
    y1_ref[...] = y1
    y2e_ref[:, 0:F] = y2
    s2 = jnp.dot(y2, a2T, preferred_element_type=jnp.float32)
    y2e_ref[:, F:W] = jnp.concatenate(
        [s2, jnp.ones((N, 1), jnp.float32),
         jnp.zeros((N, W - F - 2), jnp.float32)], axis=1)
    s1_ref[...] = jnp.dot(y1, a2T, preferred_element_type=jnp.float32)
    u3_ref[...] = jnp.dot(aT3, a2T, preferred_element_type=jnp.float32)


def _dense_pre(x, aT, a2T):
    return pl.pallas_call(
        _pre_body,
        out_shape=[
            jax.ShapeDtypeStruct((N, F), jnp.float32),
            jax.ShapeDtypeStruct((N, W), jnp.float32),
            jax.ShapeDtypeStruct((N, 1), jnp.float32),
            jax.ShapeDtypeStruct((NR, 1), jnp.float32),
        ],
    )(x, aT, a2T)


def _sc_body(sd_h, ee_h, s1_h, u3_h, y2e_h, zacc_h,
             pacc_h,
             u3_v, sd, six, eeb, pay, sv1, evb, acc,
             sdsem, eesem, rsem, g1sem, ssem):
    c = lax.axis_index("c")
    s = lax.axis_index("s")
    w = c * NS + s

    pltpu.sync_copy(zacc_h.at[pl.ds(s * RPT, RPT)], acc.at[pl.ds(s * RPT, RPT)])
    pltpu.sync_copy(u3_h, u3_v)
    plsc.subcore_barrier()

    u3vec = u3_v[...]
    u3b = [jnp.full((16,), u3vec[r], jnp.float32) for r in range(NR)]
    iota16 = lax.iota(jnp.int32, 16)
    cnt = jnp.where(c == 0, NCH0, NCH1)
    cbase = jnp.where(c == 0, s * NCH0, NS * NCH0 + s * NCH1)

    def issue_meta(j, b):
        pltpu.async_copy(sd_h.at[pl.ds(cbase + j, 1)], sd[b], sdsem[b])
        pltpu.async_copy(ee_h.at[pl.ds((cbase + j) * CHUNK * NR, CHUNK * NR)],
                         eeb[b], eesem[b])

    def wait_meta(b):
        pltpu.make_async_copy(sd_h.at[pl.ds(0, 1)], sd[b], sdsem[b]).wait()

    def issue_gathers(b):
        pltpu.async_copy(y2e_h.at[sd[b].at[0].at[1]], pay[b], rsem[b])
        pltpu.async_copy(s1_h.at[sd[b].at[0].at[0]], sv1[b], g1sem[b])

    def wait_scatter(b):
        pltpu.make_async_copy(pay[b], acc.at[six[b].at[0]], ssem[b]).wait()

    def compute(k, b):
        e_base = k * CHUNK
        pltpu.make_async_copy(ee_h.at[pl.ds(0, CHUNK * NR)], eeb[b],
                              eesem[b]).wait()
        for g in range(NGRP):
            eidx = (jnp.full((16,), g * 16, jnp.int32) + iota16) * NR
            t = jnp.zeros((16,), jnp.float32)
            for r in range(NR):
                t = t + u3b[r] * plsc.load_gather(eeb[b], [eidx + r])
            evb[b][pl.ds(g * 16, 16)] = t
        pltpu.make_async_copy(s1_h.at[sd[b].at[0].at[0]], sv1[b],
                              g1sem[b]).wait()
        pltpu.make_async_copy(y2e_h.at[sd[b].at[0].at[1]], pay[b],
                              rsem[b]).wait()
        for g in range(NGRP):
            sl = pl.ds(g * 16, 16)
            eidx16 = jnp.full((16,), g * 16, jnp.int32) + iota16
            s2g = plsc.load_gather(pay[b], [eidx16, jnp.full((16,), F, jnp.int32)])
            pre = sv1[b][sl] + s2g + evb[b][sl]
            ev = jnp.exp(-jnp.maximum(pre, ALPHA * pre))
            gidx = jnp.full((16,), e_base + g * 16, jnp.int32) + iota16
            evb[b][sl] = jnp.where(gidx < E_TOT, ev, 0.0)
            six[b][0, sl] = sd[b][0, 0, sl]

        def grp(g2, carry2):
            evg = evb[b][pl.ds(g2 * 16, 16)]
            base = g2 * 16
            for j in range(16):
                e = base + j
                evv = jnp.full((16,), evg[j], jnp.float32)
                for r in range(9):
                    pay[b][e, pl.ds(r * 16, 16)] = \
                        evv * pay[b][e, pl.ds(r * 16, 16)]
                pay[b][e, pl.ds(F + 16, NR)] = evv * eeb[b][pl.ds(e * NR, NR)]
            return carry2

        lax.fori_loop(0, NGRP, grp, 0)

    def issue_scatter(b):
        pltpu.async_copy(pay[b], acc.at[six[b].at[0]], ssem[b], add=True)

    issue_meta(0, 0)
    issue_meta(1, 1)
    wait_meta(0)
    issue_gathers(0)
    wait_meta(1)
    issue_gathers(1)
    compute(cbase + 0, 0)
    issue_meta(2, 0)
    issue_scatter(0)

    def chunk_pair(kk, carry):
        for par in range(2):
            b = (1 + par) % 2
            k = 2 * kk + 1 + par
            compute(cbase + k, b)
            wait_scatter(1 - b)
            wait_meta(1 - b)
            issue_gathers(1 - b)
            issue_meta(k + 2, b)
            issue_scatter(b)
        return carry

    lax.fori_loop(0, (cnt - 2) // 2, chunk_pair, 0)

    b = 1
    compute(cbase + cnt - 1, b)
    wait_scatter(1 - b)
    issue_scatter(b)
    wait_scatter(b)
    wait_meta(0)
    pltpu.make_async_copy(ee_h.at[pl.ds(0, CHUNK * NR)], eeb[0], eesem[0]).wait()

    plsc.subcore_barrier()
    sl = pl.ds(s * RPT, RPT)
    pltpu.sync_copy(acc.at[sl], pacc_h.at[c].at[sl])


def _sc_edge(sd3, ee1d, s1, u3, y2e, zacc):
    mesh = plsc.VectorSubcoreMesh(core_axis_name="c", subcore_axis_name="s",
                                  num_cores=NC, num_subcores=NS)
    f = pl.kernel(
        _sc_body,
        out_type=jax.ShapeDtypeStruct((NC, NPAD, W), jnp.float32),
        mesh=mesh,
        compiler_params=pltpu.CompilerParams(needs_layout_passes=False,
                                             use_tc_tiling_on_sc=False),
        scratch_types=[
            pltpu.VMEM((NR,), jnp.float32),
            [pltpu.VMEM((1, 2, CHUNK), jnp.int32) for _ in range(2)],
            [pltpu.VMEM((1, CHUNK), jnp.int32) for _ in range(2)],
            [pltpu.VMEM((CHUNK * NR,), jnp.float32) for _ in range(2)],
            [pltpu.VMEM((CHUNK, W), jnp.float32) for _ in range(2)],
            [pltpu.VMEM((CHUNK,), jnp.float32) for _ in range(2)],
            [pltpu.VMEM((CHUNK,), jnp.float32) for _ in range(2)],
            pltpu.VMEM_SHARED((NPAD, W), jnp.float32),
            [pltpu.SemaphoreType.DMA for _ in range(2)],
            [pltpu.SemaphoreType.DMA for _ in range(2)],
            [pltpu.SemaphoreType.DMA for _ in range(2)],
            [pltpu.SemaphoreType.DMA for _ in range(2)],
            [pltpu.SemaphoreType.DMA for _ in range(2)],
        ],
    )
    return f(sd3, ee1d, s1, u3, y2e, zacc)


def _combine_body(pacc_ref, y1_ref, aT_ref, out_ref):
    q = pacc_ref[0, 0:N, :] + pacc_ref[1, 0:N, :]
    s2sum = q[:, 0:F]
    rs = q[:, F + 1:F + 2]
    r16 = q[:, F + 16:W]
    aT3 = aT_ref[2 * F:2 * F + NR, :]
    ra = jnp.dot(r16, aT3, preferred_element_type=jnp.float32)
    rssafe = jnp.where(rs > 0, rs, 1.0)
    h = y1_ref[...] + (s2sum + ra) / rssafe
    h = jnp.where(rs > 0, h, 0.0)
    out_ref[...] = jnp.where(h > 0, h, jnp.exp(h) - 1.0)


def _combine(pacc, y1, aT):
    return pl.pallas_call(
        _combine_body,
        out_shape=jax.ShapeDtypeStruct((N, F), jnp.float32),
    )(pacc, y1, aT)


@jax.jit
def kernel(input, edge, edge_embed, edge_list_nhop, edge_embed_nhop, a, a_2):
    x = input
    aT = a.T
    a2T = a_2.T
    src = jnp.concatenate([edge[0], edge_list_nhop[0],
                           jnp.zeros((E_PAD - E_TOT,), jnp.int32)])
    dst = jnp.concatenate([edge[1], edge_list_nhop[1],
                           jnp.zeros((E_PAD - E_TOT,), jnp.int32)])
    ee = jnp.concatenate([edge_embed, edge_embed_nhop,
                          jnp.zeros((E_PAD - E_TOT, NR), jnp.float32)], axis=0)
    sd3 = jnp.stack([src.reshape(-1, CHUNK), dst.reshape(-1, CHUNK)], axis=1)
    sd3 = jnp.concatenate([sd3, jnp.zeros((2, 2, CHUNK), jnp.int32)], axis=0)
    ee1d = jnp.concatenate([ee.reshape(E_PAD * NR),
                            jnp.zeros((2 * CHUNK * NR,), jnp.float32)])

    y1, y2e, s1, u3 = _dense_pre(x, aT, a2T)
    zacc = jnp.zeros((NPAD, W), jnp.float32)
    pacc = _sc_edge(sd3, ee1d, s1.reshape(N), u3.reshape(NR), y2e, zacc)
    return _combine(pacc, y1, aT)
